# baseline (speedup 1.0000x reference)
import functools

import jax
import jax.numpy as jnp
from jax import lax
from jax.experimental import pallas as pl
from jax.experimental.pallas import tpu as pltpu
from jax.experimental.pallas import tpu_sc as plsc

B = 4096
L = 50
NN = 20
KV = 32
MEM = 64

NC = 2
NS = 16
NW = NC * NS
PER_W = B // NW
SEQ_CB = 8
SEQ_NCHUNK = PER_W // SEQ_CB
SGL_CB = 64
NROWS_W = (B * NN) // NW
NEIGH_CB = 160
NEIGH_NCHUNK = NROWS_W // NEIGH_CB


@functools.lru_cache(maxsize=1)
def _make_sc_gather():
  mesh = plsc.VectorSubcoreMesh(
      core_axis_name="c", subcore_axis_name="s", num_cores=NC, num_subcores=NS)
  return functools.partial(
      pl.kernel,
      out_type=(
          jax.ShapeDtypeStruct((B, 128), jnp.float32),
          jax.ShapeDtypeStruct((B, 128), jnp.float32),
          jax.ShapeDtypeStruct((B, 128), jnp.float32),
          jax.ShapeDtypeStruct((B, 128), jnp.float32),
          jax.ShapeDtypeStruct((B, KV), jnp.float32),
          jax.ShapeDtypeStruct((B, KV), jnp.float32),
          jax.ShapeDtypeStruct((B * NN, 128), jnp.float32),
      ),
      mesh=mesh,
      scratch_types=[
          pltpu.VMEM((SGL_CB,), jnp.int32),
          pltpu.VMEM((SGL_CB, 128), jnp.float32),
          pltpu.VMEM((SEQ_CB * L,), jnp.int32),
          pltpu.VMEM((SEQ_CB * L, 128), jnp.float32),
          pltpu.VMEM((SEQ_CB, KV), jnp.float32),
          pltpu.VMEM((NEIGH_CB,), jnp.int32),
          pltpu.VMEM((NEIGH_CB, 128), jnp.float32),
          pltpu.SemaphoreType.DMA,
      ],
      compiler_params=pltpu.CompilerParams(use_tc_tiling_on_sc=False),
  )(_sc_gather_body)


def _sc_gather_body(age_tab, gen_tab, iid_tab, icat_tab, mem1_tab,
                    age_idx, gen_idx, iid_idx, icat_idx,
                    idseq, cateff, nidx,
                    age_out, gen_out, iid_out, icat_out, sumid_out, sumcat_out,
                    neigh_out,
                    sidx_v, srows_v, qidx_v, qrows_v, qsum_v, nidx_v, nrows_v,
                    sem):
  wid = lax.axis_index("s") * NC + lax.axis_index("c")
  base = wid * PER_W

  for tab, idx_hbm, out_hbm in (
      (age_tab, age_idx, age_out),
      (gen_tab, gen_idx, gen_out),
      (iid_tab, iid_idx, iid_out),
      (icat_tab, icat_idx, icat_out),
  ):
    for c in range(PER_W // SGL_CB):
      off = base + c * SGL_CB
      pltpu.sync_copy(idx_hbm.at[pl.ds(off, SGL_CB)], sidx_v)
      pltpu.async_copy(tab.at[sidx_v], srows_v, sem).wait()
      pltpu.sync_copy(srows_v, out_hbm.at[pl.ds(off, SGL_CB)])

  for tab, idx_hbm, out_hbm in ((iid_tab, idseq, sumid_out),
                                (icat_tab, cateff, sumcat_out)):
    def seq_chunk(c, tab=tab, idx_hbm=idx_hbm, out_hbm=out_hbm):
      roff = base * L + c * (SEQ_CB * L)
      pltpu.sync_copy(idx_hbm.at[pl.ds(roff, SEQ_CB * L)], qidx_v)
      pltpu.async_copy(tab.at[qidx_v], qrows_v, sem).wait()

      def accum_b(b, _):
        r0 = b * L

        def accum_l(l, accs):
          a0, a1 = accs
          return (a0 + qrows_v[r0 + l, pl.ds(0, 16)],
                  a1 + qrows_v[r0 + l, pl.ds(16, 16)])

        z = jnp.zeros((16,), jnp.float32)
        a0, a1 = lax.fori_loop(0, L, accum_l, (z, z), unroll=10)
        qsum_v[b, pl.ds(0, 16)] = a0
        qsum_v[b, pl.ds(16, 16)] = a1
        return 0

      lax.fori_loop(0, SEQ_CB, accum_b, 0)
      pltpu.sync_copy(qsum_v, out_hbm.at[pl.ds(base + c * SEQ_CB, SEQ_CB)])
      return 0

    lax.fori_loop(0, SEQ_NCHUNK, lambda c, _, f=seq_chunk: f(c), 0)

  nbase = wid * NROWS_W

  def neigh_chunk(c, _):
    roff = nbase + c * NEIGH_CB
    pltpu.sync_copy(nidx.at[pl.ds(roff, NEIGH_CB)], nidx_v)
    pltpu.async_copy(mem1_tab.at[nidx_v], nrows_v, sem).wait()
    pltpu.sync_copy(nrows_v, neigh_out.at[pl.ds(roff, NEIGH_CB)])
    return 0

  lax.fori_loop(0, NEIGH_NCHUNK, neigh_chunk, 0)


BS = 256


def _tc_body(idseq_ref, nidt_ref, age_ref, gen_ref, iid_ref, icat_ref,
             sumid_ref, sumcat_ref, neigh_ref,
             wagg_ref, bagg_ref, wp0_ref, bp0_ref, wp1_ref, bp1_ref,
             wp2_ref, bp2_ref, out_ref):
  f32 = jnp.float32
  cnt = jnp.sum((idseq_ref[...] != 0).astype(f32), axis=1)
  den = jnp.maximum(cnt, 1.0)[:, None]
  sm_id = sumid_ref[...] / den
  sm_cat = sumcat_ref[...] / den

  nm = (nidt_ref[...] != 0).astype(f32)
  cntn = jnp.maximum(jnp.sum(nm, axis=0), 1.0)
  W = wagg_ref[...]
  bb = bagg_ref[...]
  acc = jnp.zeros((BS, MEM), f32)
  for n in range(NN):
    h = jnp.tanh(
        jnp.dot(neigh_ref[n][:, :MEM], W, preferred_element_type=f32)
        + bb[None, :])
    acc = acc + h * nm[n][:, None]
  gnn = acc / cntn[:, None]

  iid_e = iid_ref[...][:, :KV]
  icat_e = icat_ref[...][:, :KV]
  feat = jnp.concatenate(
      [age_ref[...][:, :KV], gen_ref[...][:, :KV], iid_e, icat_e, sm_id,
       sm_cat, iid_e * sm_id, icat_e * sm_cat, gnn], axis=1)
  h0 = jnp.maximum(
      jnp.dot(feat, wp0_ref[...], preferred_element_type=f32)
      + bp0_ref[...][None, :], 0.0)
  h1 = jnp.maximum(
      jnp.dot(h0, wp1_ref[...], preferred_element_type=f32)
      + bp1_ref[...][None, :], 0.0)
  logit = jnp.dot(h1, wp2_ref[...], preferred_element_type=f32) + bp2_ref[0]
  out_ref[...] = 1.0 / (1.0 + jnp.exp(-logit))


def _tc_call(idseq, nid_t, age_e, gen_e, iid_e, icat_e, sum_id, sum_cat,
             neigh_t, W_agg1, b_agg1, W_p0, b_p0, W_p1, b_p1, W_p2, b_p2):
  nblk = B // BS
  bcast = lambda shape: pl.BlockSpec(shape, lambda i: tuple(0 for _ in shape))
  return pl.pallas_call(
      _tc_body,
      grid=(nblk,),
      in_specs=[
          pl.BlockSpec((BS, L), lambda i: (i, 0)),
          pl.BlockSpec((NN, BS), lambda i: (0, i)),
          pl.BlockSpec((BS, 128), lambda i: (i, 0)),
          pl.BlockSpec((BS, 128), lambda i: (i, 0)),
          pl.BlockSpec((BS, 128), lambda i: (i, 0)),
          pl.BlockSpec((BS, 128), lambda i: (i, 0)),
          pl.BlockSpec((BS, KV), lambda i: (i, 0)),
          pl.BlockSpec((BS, KV), lambda i: (i, 0)),
          pl.BlockSpec((NN, BS, 128), lambda i: (0, i, 0)),
          bcast((MEM, MEM)),
          bcast((MEM,)),
          bcast((5 * MEM, 256)),
          bcast((256,)),
          bcast((256, 128)),
          bcast((128,)),
          bcast((128, 1)),
          bcast((1,)),
      ],
      out_specs=pl.BlockSpec((BS, 1), lambda i: (i, 0)),
      out_shape=jax.ShapeDtypeStruct((B, 1), jnp.float32),
  )(idseq, nid_t, age_e, gen_e, iid_e, icat_e, sum_id, sum_cat, neigh_t,
    W_agg1, b_agg1, W_p0, b_p0, W_p1, b_p1, W_p2, b_p2)


def _pad128(t):
  return jnp.pad(t, ((0, 0), (0, 128 - t.shape[1])))


def kernel(userid, user_age, user_gender, item_id, item_cate, item_id_seq,
           item_cate_seq, neighbor_ids, user_age_table, user_gender_table,
           item_id_table, item_cate_table, user_mem_0, user_mem_1,
           W_agg0, b_agg0, W_agg1, b_agg1,
           W_p0, b_p0, W_p1, b_p1, W_p2, b_p2):
  i32 = jnp.int32
  idseq = item_id_seq.astype(i32)
  cateff = jnp.where(idseq == 0, 0, item_cate_seq.astype(i32))
  nid = neighbor_ids.astype(i32)
  nid_t = nid.T

  (age_e, gen_e, iid_e, icat_e, sum_id, sum_cat, neigh_rows) = _make_sc_gather()(
      _pad128(user_age_table), _pad128(user_gender_table),
      _pad128(item_id_table), _pad128(item_cate_table),
      _pad128(user_mem_1),
      user_age.astype(i32), user_gender.astype(i32),
      item_id.astype(i32), item_cate.astype(i32),
      idseq.reshape(-1), cateff.reshape(-1), nid_t.reshape(-1))

  out = _tc_call(
      idseq, nid_t, age_e, gen_e, iid_e, icat_e, sum_id, sum_cat,
      neigh_rows.reshape(NN, B, 128),
      W_agg1, b_agg1, W_p0, b_p0, W_p1, b_p1, W_p2, b_p2)
  return out[:, 0]


# speedup vs baseline: 1.0157x; 1.0157x over previous
import functools

import jax
import jax.numpy as jnp
from jax import lax
from jax.experimental import pallas as pl
from jax.experimental.pallas import tpu as pltpu
from jax.experimental.pallas import tpu_sc as plsc

B = 4096
L = 50
NN = 20
KV = 32
MEM = 64

NC = 2
NS = 16
NW = NC * NS
PER_W = B // NW
SEQ_CB = 8
SEQ_NCHUNK = PER_W // SEQ_CB
SGL_CB = 64
NROWS_W = (B * NN) // NW
NEIGH_CB = 160
NEIGH_NCHUNK = NROWS_W // NEIGH_CB


@functools.lru_cache(maxsize=1)
def _make_sc_gather():
  mesh = plsc.VectorSubcoreMesh(
      core_axis_name="c", subcore_axis_name="s", num_cores=NC, num_subcores=NS)
  return functools.partial(
      pl.kernel,
      out_type=(
          jax.ShapeDtypeStruct((B, 128), jnp.float32),
          jax.ShapeDtypeStruct((B, 128), jnp.float32),
          jax.ShapeDtypeStruct((B, 128), jnp.float32),
          jax.ShapeDtypeStruct((B, 128), jnp.float32),
          jax.ShapeDtypeStruct((B, KV), jnp.float32),
          jax.ShapeDtypeStruct((B, KV), jnp.float32),
          jax.ShapeDtypeStruct((B * NN, 128), jnp.float32),
      ),
      mesh=mesh,
      scratch_types=[
          pltpu.VMEM((SGL_CB,), jnp.int32),
          pltpu.VMEM((SGL_CB, 128), jnp.float32),
          pltpu.VMEM((SEQ_CB * L,), jnp.int32),
          pltpu.VMEM((SEQ_CB * L, 128), jnp.float32),
          pltpu.VMEM((SEQ_CB, KV), jnp.float32),
          pltpu.VMEM((NEIGH_CB,), jnp.int32),
          pltpu.VMEM((NEIGH_CB, 128), jnp.float32),
          pltpu.SemaphoreType.DMA,
      ],
      compiler_params=pltpu.CompilerParams(use_tc_tiling_on_sc=False),
  )(_sc_gather_body)


def _sc_gather_body(age_tab, gen_tab, iid_tab, icat_tab, mem1_tab,
                    age_idx, gen_idx, iid_idx, icat_idx,
                    idseq, cateff, nidx,
                    age_out, gen_out, iid_out, icat_out, sumid_out, sumcat_out,
                    neigh_out,
                    sidx_v, srows_v, qidx_v, qrows_v, qsum_v, nidx_v, nrows_v,
                    sem):
  wid = lax.axis_index("s") * NC + lax.axis_index("c")
  base = wid * PER_W

  for tab, idx_hbm, out_hbm in (
      (age_tab, age_idx, age_out),
      (gen_tab, gen_idx, gen_out),
      (iid_tab, iid_idx, iid_out),
      (icat_tab, icat_idx, icat_out),
  ):
    for c in range(PER_W // SGL_CB):
      off = base + c * SGL_CB
      pltpu.sync_copy(idx_hbm.at[pl.ds(off, SGL_CB)], sidx_v)
      pltpu.async_copy(tab.at[sidx_v], srows_v, sem).wait()
      pltpu.sync_copy(srows_v, out_hbm.at[pl.ds(off, SGL_CB)])

  for tab, idx_hbm, out_hbm in ((iid_tab, idseq, sumid_out),
                                (icat_tab, cateff, sumcat_out)):
    def seq_chunk(c, tab=tab, idx_hbm=idx_hbm, out_hbm=out_hbm):
      roff = base * L + c * (SEQ_CB * L)
      pltpu.sync_copy(idx_hbm.at[pl.ds(roff, SEQ_CB * L)], qidx_v)
      pltpu.async_copy(tab.at[qidx_v], qrows_v, sem).wait()

      def accum_b(b, _):
        r0 = b * L

        def accum_l(l, accs):
          a0, a1 = accs
          return (a0 + qrows_v[r0 + l, pl.ds(0, 16)],
                  a1 + qrows_v[r0 + l, pl.ds(16, 16)])

        z = jnp.zeros((16,), jnp.float32)
        a0, a1 = lax.fori_loop(0, L, accum_l, (z, z), unroll=10)
        qsum_v[b, pl.ds(0, 16)] = a0
        qsum_v[b, pl.ds(16, 16)] = a1
        return 0

      lax.fori_loop(0, SEQ_CB, accum_b, 0)
      pltpu.sync_copy(qsum_v, out_hbm.at[pl.ds(base + c * SEQ_CB, SEQ_CB)])
      return 0

    lax.fori_loop(0, SEQ_NCHUNK, lambda c, _, f=seq_chunk: f(c), 0)

  nbase = wid * NROWS_W

  def neigh_chunk(c, _):
    roff = nbase + c * NEIGH_CB
    pltpu.sync_copy(nidx.at[pl.ds(roff, NEIGH_CB)], nidx_v)
    pltpu.async_copy(mem1_tab.at[nidx_v], nrows_v, sem).wait()
    pltpu.sync_copy(nrows_v, neigh_out.at[pl.ds(roff, NEIGH_CB)])
    return 0

  lax.fori_loop(0, NEIGH_NCHUNK, neigh_chunk, 0)


BS = 256


def _tc_body(idseq_ref, nidt_ref, age_ref, gen_ref, iid_ref, icat_ref,
             sumid_ref, sumcat_ref, neigh_ref,
             wagg_ref, bagg_ref, wp0_ref, bp0_ref, wp1_ref, bp1_ref,
             wp2_ref, bp2_ref, out_ref):
  f32 = jnp.float32
  cnt = jnp.sum((idseq_ref[...] != 0).astype(f32), axis=1)
  den = jnp.maximum(cnt, 1.0)[:, None]
  sm_id = sumid_ref[...] / den
  sm_cat = sumcat_ref[...] / den

  nm = (nidt_ref[...] != 0).astype(f32)
  cntn = jnp.maximum(jnp.sum(nm, axis=0), 1.0)
  W = wagg_ref[...]
  bb = bagg_ref[...]
  acc = jnp.zeros((BS, MEM), f32)
  for n in range(NN):
    h = jnp.tanh(
        jnp.dot(neigh_ref[n][:, :MEM], W, preferred_element_type=f32)
        + bb[None, :])
    acc = acc + h * nm[n][:, None]
  gnn = acc / cntn[:, None]

  iid_e = iid_ref[...][:, :KV]
  icat_e = icat_ref[...][:, :KV]
  feat = jnp.concatenate(
      [age_ref[...][:, :KV], gen_ref[...][:, :KV], iid_e, icat_e, sm_id,
       sm_cat, iid_e * sm_id, icat_e * sm_cat, gnn], axis=1)
  h0 = jnp.maximum(
      jnp.dot(feat, wp0_ref[...], preferred_element_type=f32)
      + bp0_ref[...][None, :], 0.0)
  h1 = jnp.maximum(
      jnp.dot(h0, wp1_ref[...], preferred_element_type=f32)
      + bp1_ref[...][None, :], 0.0)
  logit = jnp.dot(h1, wp2_ref[...], preferred_element_type=f32) + bp2_ref[0]
  out_ref[...] = 1.0 / (1.0 + jnp.exp(-logit))


def _tc_call(idseq, nid_t, age_e, gen_e, iid_e, icat_e, sum_id, sum_cat,
             neigh_t, W_agg1, b_agg1, W_p0, b_p0, W_p1, b_p1, W_p2, b_p2):
  nblk = B // BS
  bcast = lambda shape: pl.BlockSpec(shape, lambda i: tuple(0 for _ in shape))
  return pl.pallas_call(
      _tc_body,
      grid=(nblk,),
      in_specs=[
          pl.BlockSpec((BS, L), lambda i: (i, 0)),
          pl.BlockSpec((NN, BS), lambda i: (0, i)),
          pl.BlockSpec((BS, 128), lambda i: (i, 0)),
          pl.BlockSpec((BS, 128), lambda i: (i, 0)),
          pl.BlockSpec((BS, 128), lambda i: (i, 0)),
          pl.BlockSpec((BS, 128), lambda i: (i, 0)),
          pl.BlockSpec((BS, KV), lambda i: (i, 0)),
          pl.BlockSpec((BS, KV), lambda i: (i, 0)),
          pl.BlockSpec((NN, BS, 128), lambda i: (0, i, 0)),
          bcast((MEM, MEM)),
          bcast((MEM,)),
          bcast((5 * MEM, 256)),
          bcast((256,)),
          bcast((256, 128)),
          bcast((128,)),
          bcast((128, 1)),
          bcast((1,)),
      ],
      out_specs=pl.BlockSpec((BS, 1), lambda i: (i, 0)),
      out_shape=jax.ShapeDtypeStruct((B, 1), jnp.float32),
  )(idseq, nid_t, age_e, gen_e, iid_e, icat_e, sum_id, sum_cat, neigh_t,
    W_agg1, b_agg1, W_p0, b_p0, W_p1, b_p1, W_p2, b_p2)


def _tr_body(x_ref, o_ref):
  xt = jnp.transpose(x_ref[...])
  o_ref[...] = jnp.pad(xt, ((0, 0), (0, 128 - xt.shape[1])))


def _to_rows128(t):
  v, r = t.shape
  blk = 2048 if v >= 2048 else v
  return pl.pallas_call(
      _tr_body,
      grid=(pl.cdiv(v, blk),),
      in_specs=[pl.BlockSpec((r, blk), lambda i: (0, i))],
      out_specs=pl.BlockSpec((blk, 128), lambda i: (i, 0)),
      out_shape=jax.ShapeDtypeStruct((v, 128), jnp.float32),
  )(t.T)


def kernel(userid, user_age, user_gender, item_id, item_cate, item_id_seq,
           item_cate_seq, neighbor_ids, user_age_table, user_gender_table,
           item_id_table, item_cate_table, user_mem_0, user_mem_1,
           W_agg0, b_agg0, W_agg1, b_agg1,
           W_p0, b_p0, W_p1, b_p1, W_p2, b_p2):
  i32 = jnp.int32
  idseq = item_id_seq.astype(i32)
  cateff = jnp.where(idseq == 0, 0, item_cate_seq.astype(i32))
  nid = neighbor_ids.astype(i32)
  nid_t = nid.T

  (age_e, gen_e, iid_e, icat_e, sum_id, sum_cat, neigh_rows) = _make_sc_gather()(
      _to_rows128(user_age_table), _to_rows128(user_gender_table),
      _to_rows128(item_id_table), _to_rows128(item_cate_table),
      _to_rows128(user_mem_1),
      user_age.astype(i32), user_gender.astype(i32),
      item_id.astype(i32), item_cate.astype(i32),
      idseq.reshape(-1), cateff.reshape(-1), nid_t.reshape(-1))

  out = _tc_call(
      idseq, nid_t, age_e, gen_e, iid_e, icat_e, sum_id, sum_cat,
      neigh_rows.reshape(NN, B, 128),
      W_agg1, b_agg1, W_p0, b_p0, W_p1, b_p1, W_p2, b_p2)
  return out[:, 0]


# speedup vs baseline: 1.0604x; 1.0440x over previous
import functools

import jax
import jax.numpy as jnp
from jax import lax
from jax.experimental import pallas as pl
from jax.experimental.pallas import tpu as pltpu
from jax.experimental.pallas import tpu_sc as plsc

B = 4096
L = 50
NN = 20
KV = 32
MEM = 64

NC = 2
NS = 16
NW = NC * NS
PER_W = B // NW
SEQ_CB = 8
SEQ_NCHUNK = PER_W // SEQ_CB
SGL_CB = 64
NROWS_W = (B * NN) // NW
NEIGH_CB = 160
NEIGH_NCHUNK = NROWS_W // NEIGH_CB


@functools.lru_cache(maxsize=1)
def _make_sc_gather():
  mesh = plsc.VectorSubcoreMesh(
      core_axis_name="c", subcore_axis_name="s", num_cores=NC, num_subcores=NS)
  return functools.partial(
      pl.kernel,
      out_type=(
          jax.ShapeDtypeStruct((B, KV), jnp.float32),
          jax.ShapeDtypeStruct((B, KV), jnp.float32),
          jax.ShapeDtypeStruct((B, KV), jnp.float32),
          jax.ShapeDtypeStruct((B, KV), jnp.float32),
          jax.ShapeDtypeStruct((B, KV), jnp.float32),
          jax.ShapeDtypeStruct((B, KV), jnp.float32),
          jax.ShapeDtypeStruct((B * NN, MEM), jnp.float32),
      ),
      mesh=mesh,
      scratch_types=[
          pltpu.VMEM((SGL_CB,), jnp.int32),
          pltpu.VMEM((SGL_CB, KV), jnp.float32),
          pltpu.VMEM((SEQ_CB * L,), jnp.int32),
          pltpu.VMEM((SEQ_CB * L, KV), jnp.float32),
          pltpu.VMEM((SEQ_CB, KV), jnp.float32),
          pltpu.VMEM((NEIGH_CB,), jnp.int32),
          pltpu.VMEM((NEIGH_CB, MEM), jnp.float32),
          pltpu.SemaphoreType.DMA,
      ],
      compiler_params=pltpu.CompilerParams(use_tc_tiling_on_sc=False),
  )(_sc_gather_body)


def _sc_gather_body(age_tab, gen_tab, iid_tab, icat_tab, mem1_tab,
                    age_idx, gen_idx, iid_idx, icat_idx,
                    idseq, cateff, nidx,
                    age_out, gen_out, iid_out, icat_out, sumid_out, sumcat_out,
                    neigh_out,
                    sidx_v, srows_v, qidx_v, qrows_v, qsum_v, nidx_v, nrows_v,
                    sem):
  wid = lax.axis_index("s") * NC + lax.axis_index("c")
  base = wid * PER_W

  for tab, idx_hbm, out_hbm in (
      (age_tab, age_idx, age_out),
      (gen_tab, gen_idx, gen_out),
      (iid_tab, iid_idx, iid_out),
      (icat_tab, icat_idx, icat_out),
  ):
    for c in range(PER_W // SGL_CB):
      off = base + c * SGL_CB
      pltpu.sync_copy(idx_hbm.at[pl.ds(off, SGL_CB)], sidx_v)
      pltpu.async_copy(tab.at[sidx_v], srows_v, sem).wait()
      pltpu.sync_copy(srows_v, out_hbm.at[pl.ds(off, SGL_CB)])

  for tab, idx_hbm, out_hbm in ((iid_tab, idseq, sumid_out),
                                (icat_tab, cateff, sumcat_out)):
    def seq_chunk(c, tab=tab, idx_hbm=idx_hbm, out_hbm=out_hbm):
      roff = base * L + c * (SEQ_CB * L)
      pltpu.sync_copy(idx_hbm.at[pl.ds(roff, SEQ_CB * L)], qidx_v)
      pltpu.async_copy(tab.at[qidx_v], qrows_v, sem).wait()

      def accum_b(b, _):
        r0 = b * L

        def accum_l(l, accs):
          a0, a1 = accs
          return (a0 + qrows_v[r0 + l, pl.ds(0, 16)],
                  a1 + qrows_v[r0 + l, pl.ds(16, 16)])

        z = jnp.zeros((16,), jnp.float32)
        a0, a1 = lax.fori_loop(0, L, accum_l, (z, z), unroll=10)
        qsum_v[b, pl.ds(0, 16)] = a0
        qsum_v[b, pl.ds(16, 16)] = a1
        return 0

      lax.fori_loop(0, SEQ_CB, accum_b, 0)
      pltpu.sync_copy(qsum_v, out_hbm.at[pl.ds(base + c * SEQ_CB, SEQ_CB)])
      return 0

    lax.fori_loop(0, SEQ_NCHUNK, lambda c, _, f=seq_chunk: f(c), 0)

  nbase = wid * NROWS_W

  def neigh_chunk(c, _):
    roff = nbase + c * NEIGH_CB
    pltpu.sync_copy(nidx.at[pl.ds(roff, NEIGH_CB)], nidx_v)
    pltpu.async_copy(mem1_tab.at[nidx_v], nrows_v, sem).wait()
    pltpu.sync_copy(nrows_v, neigh_out.at[pl.ds(roff, NEIGH_CB)])
    return 0

  lax.fori_loop(0, NEIGH_NCHUNK, neigh_chunk, 0)


BS = 256


def _tc_body(idseq_ref, nidt_ref, age_ref, gen_ref, iid_ref, icat_ref,
             sumid_ref, sumcat_ref, neigh_ref,
             wagg_ref, bagg_ref, wp0_ref, bp0_ref, wp1_ref, bp1_ref,
             wp2_ref, bp2_ref, out_ref):
  f32 = jnp.float32
  cnt = jnp.sum((idseq_ref[...] != 0).astype(f32), axis=1)
  den = jnp.maximum(cnt, 1.0)[:, None]
  sm_id = sumid_ref[...] / den
  sm_cat = sumcat_ref[...] / den

  nm = (nidt_ref[...] != 0).astype(f32)
  cntn = jnp.maximum(jnp.sum(nm, axis=0), 1.0)
  W = wagg_ref[...]
  bb = bagg_ref[...]
  acc = jnp.zeros((BS, MEM), f32)
  for n in range(NN):
    h = jnp.tanh(
        jnp.dot(neigh_ref[n], W, preferred_element_type=f32)
        + bb[None, :])
    acc = acc + h * nm[n][:, None]
  gnn = acc / cntn[:, None]

  iid_e = iid_ref[...]
  icat_e = icat_ref[...]
  feat = jnp.concatenate(
      [age_ref[...], gen_ref[...], iid_e, icat_e, sm_id,
       sm_cat, iid_e * sm_id, icat_e * sm_cat, gnn], axis=1)
  h0 = jnp.maximum(
      jnp.dot(feat, wp0_ref[...], preferred_element_type=f32)
      + bp0_ref[...][None, :], 0.0)
  h1 = jnp.maximum(
      jnp.dot(h0, wp1_ref[...], preferred_element_type=f32)
      + bp1_ref[...][None, :], 0.0)
  logit = jnp.dot(h1, wp2_ref[...], preferred_element_type=f32) + bp2_ref[0]
  out_ref[...] = 1.0 / (1.0 + jnp.exp(-logit))


def _tc_call(idseq, nid_t, age_e, gen_e, iid_e, icat_e, sum_id, sum_cat,
             neigh_t, W_agg1, b_agg1, W_p0, b_p0, W_p1, b_p1, W_p2, b_p2):
  nblk = B // BS
  bcast = lambda shape: pl.BlockSpec(shape, lambda i: tuple(0 for _ in shape))
  return pl.pallas_call(
      _tc_body,
      grid=(nblk,),
      in_specs=[
          pl.BlockSpec((BS, L), lambda i: (i, 0)),
          pl.BlockSpec((NN, BS), lambda i: (0, i)),
          pl.BlockSpec((BS, KV), lambda i: (i, 0)),
          pl.BlockSpec((BS, KV), lambda i: (i, 0)),
          pl.BlockSpec((BS, KV), lambda i: (i, 0)),
          pl.BlockSpec((BS, KV), lambda i: (i, 0)),
          pl.BlockSpec((BS, KV), lambda i: (i, 0)),
          pl.BlockSpec((BS, KV), lambda i: (i, 0)),
          pl.BlockSpec((NN, BS, MEM), lambda i: (0, i, 0)),
          bcast((MEM, MEM)),
          bcast((MEM,)),
          bcast((5 * MEM, 256)),
          bcast((256,)),
          bcast((256, 128)),
          bcast((128,)),
          bcast((128, 1)),
          bcast((1,)),
      ],
      out_specs=pl.BlockSpec((BS, 1), lambda i: (i, 0)),
      out_shape=jax.ShapeDtypeStruct((B, 1), jnp.float32),
  )(idseq, nid_t, age_e, gen_e, iid_e, icat_e, sum_id, sum_cat, neigh_t,
    W_agg1, b_agg1, W_p0, b_p0, W_p1, b_p1, W_p2, b_p2)


def kernel(userid, user_age, user_gender, item_id, item_cate, item_id_seq,
           item_cate_seq, neighbor_ids, user_age_table, user_gender_table,
           item_id_table, item_cate_table, user_mem_0, user_mem_1,
           W_agg0, b_agg0, W_agg1, b_agg1,
           W_p0, b_p0, W_p1, b_p1, W_p2, b_p2):
  i32 = jnp.int32
  idseq = item_id_seq.astype(i32)
  cateff = jnp.where(idseq == 0, 0, item_cate_seq.astype(i32))
  nid = neighbor_ids.astype(i32)
  nid_t = nid.T

  (age_e, gen_e, iid_e, icat_e, sum_id, sum_cat, neigh_rows) = _make_sc_gather()(
      user_age_table, user_gender_table, item_id_table, item_cate_table,
      user_mem_1,
      user_age.astype(i32), user_gender.astype(i32),
      item_id.astype(i32), item_cate.astype(i32),
      idseq.reshape(-1), cateff.reshape(-1), nid_t.reshape(-1))

  out = _tc_call(
      idseq, nid_t, age_e, gen_e, iid_e, icat_e, sum_id, sum_cat,
      neigh_rows.reshape(NN, B, MEM),
      W_agg1, b_agg1, W_p0, b_p0, W_p1, b_p1, W_p2, b_p2)
  return out[:, 0]


# speedup vs baseline: 1.1050x; 1.0421x over previous
import functools

import jax
import jax.numpy as jnp
from jax import lax
from jax.experimental import pallas as pl
from jax.experimental.pallas import tpu as pltpu
from jax.experimental.pallas import tpu_sc as plsc

B = 4096
L = 50
NN = 20
KV = 32
MEM = 64

NC = 2
NS = 16
NW = NC * NS
PER_W = B // NW
SEQ_CB = 16
SEQ_NCHUNK = PER_W // SEQ_CB
SGL_CB = 128
NROWS_W = (B * NN) // NW
NEIGH_CB = 320
NEIGH_NCHUNK = NROWS_W // NEIGH_CB


@functools.lru_cache(maxsize=1)
def _make_sc_gather():
  mesh = plsc.VectorSubcoreMesh(
      core_axis_name="c", subcore_axis_name="s", num_cores=NC, num_subcores=NS)
  return functools.partial(
      pl.kernel,
      out_type=(
          jax.ShapeDtypeStruct((B, KV), jnp.float32),
          jax.ShapeDtypeStruct((B, KV), jnp.float32),
          jax.ShapeDtypeStruct((B, KV), jnp.float32),
          jax.ShapeDtypeStruct((B, KV), jnp.float32),
          jax.ShapeDtypeStruct((B, KV), jnp.float32),
          jax.ShapeDtypeStruct((B, KV), jnp.float32),
          jax.ShapeDtypeStruct((B * NN, MEM), jnp.float32),
      ),
      mesh=mesh,
      scratch_types=[
          pltpu.VMEM((SGL_CB,), jnp.int32),
          pltpu.VMEM((SGL_CB, KV), jnp.float32),
          pltpu.VMEM((SEQ_CB * L,), jnp.int32),
          pltpu.VMEM((SEQ_CB * L, KV), jnp.float32),
          pltpu.VMEM((SEQ_CB, KV), jnp.float32),
          pltpu.VMEM((NEIGH_CB,), jnp.int32),
          pltpu.VMEM((NEIGH_CB, MEM), jnp.float32),
          pltpu.SemaphoreType.DMA,
      ],
      compiler_params=pltpu.CompilerParams(use_tc_tiling_on_sc=False),
  )(_sc_gather_body)


def _sc_gather_body(age_tab, gen_tab, iid_tab, icat_tab, mem1_tab,
                    age_idx, gen_idx, iid_idx, icat_idx,
                    idseq, cateff, nidx,
                    age_out, gen_out, iid_out, icat_out, sumid_out, sumcat_out,
                    neigh_out,
                    sidx_v, srows_v, qidx_v, qrows_v, qsum_v, nidx_v, nrows_v,
                    sem):
  wid = lax.axis_index("s") * NC + lax.axis_index("c")
  base = wid * PER_W

  for tab, idx_hbm, out_hbm in (
      (age_tab, age_idx, age_out),
      (gen_tab, gen_idx, gen_out),
      (iid_tab, iid_idx, iid_out),
      (icat_tab, icat_idx, icat_out),
  ):
    for c in range(PER_W // SGL_CB):
      off = base + c * SGL_CB
      pltpu.sync_copy(idx_hbm.at[pl.ds(off, SGL_CB)], sidx_v)
      pltpu.async_copy(tab.at[sidx_v], srows_v, sem).wait()
      pltpu.sync_copy(srows_v, out_hbm.at[pl.ds(off, SGL_CB)])

  for tab, idx_hbm, out_hbm in ((iid_tab, idseq, sumid_out),
                                (icat_tab, cateff, sumcat_out)):
    def seq_chunk(c, tab=tab, idx_hbm=idx_hbm, out_hbm=out_hbm):
      roff = base * L + c * (SEQ_CB * L)
      pltpu.sync_copy(idx_hbm.at[pl.ds(roff, SEQ_CB * L)], qidx_v)
      pltpu.async_copy(tab.at[qidx_v], qrows_v, sem).wait()

      def accum_b(b, _):
        r0 = b * L

        def accum_l(l, accs):
          a0, a1 = accs
          return (a0 + qrows_v[r0 + l, pl.ds(0, 16)],
                  a1 + qrows_v[r0 + l, pl.ds(16, 16)])

        z = jnp.zeros((16,), jnp.float32)
        a0, a1 = lax.fori_loop(0, L, accum_l, (z, z), unroll=10)
        qsum_v[b, pl.ds(0, 16)] = a0
        qsum_v[b, pl.ds(16, 16)] = a1
        return 0

      lax.fori_loop(0, SEQ_CB, accum_b, 0)
      pltpu.sync_copy(qsum_v, out_hbm.at[pl.ds(base + c * SEQ_CB, SEQ_CB)])
      return 0

    lax.fori_loop(0, SEQ_NCHUNK, lambda c, _, f=seq_chunk: f(c), 0)

  nbase = wid * NROWS_W

  def neigh_chunk(c, _):
    roff = nbase + c * NEIGH_CB
    pltpu.sync_copy(nidx.at[pl.ds(roff, NEIGH_CB)], nidx_v)
    pltpu.async_copy(mem1_tab.at[nidx_v], nrows_v, sem).wait()
    pltpu.sync_copy(nrows_v, neigh_out.at[pl.ds(roff, NEIGH_CB)])
    return 0

  lax.fori_loop(0, NEIGH_NCHUNK, neigh_chunk, 0)


BS = 256


def _tc_body(idseq_ref, nidt_ref, age_ref, gen_ref, iid_ref, icat_ref,
             sumid_ref, sumcat_ref, neigh_ref,
             wagg_ref, bagg_ref, wp0_ref, bp0_ref, wp1_ref, bp1_ref,
             wp2_ref, bp2_ref, out_ref):
  f32 = jnp.float32
  cnt = jnp.sum((idseq_ref[...] != 0).astype(f32), axis=1)
  den = jnp.maximum(cnt, 1.0)[:, None]
  sm_id = sumid_ref[...] / den
  sm_cat = sumcat_ref[...] / den

  nm = (nidt_ref[...] != 0).astype(f32)
  cntn = jnp.maximum(jnp.sum(nm, axis=0), 1.0)
  W = wagg_ref[...]
  bb = bagg_ref[...]
  acc = jnp.zeros((BS, MEM), f32)
  for n in range(NN):
    h = jnp.tanh(
        jnp.dot(neigh_ref[n], W, preferred_element_type=f32)
        + bb[None, :])
    acc = acc + h * nm[n][:, None]
  gnn = acc / cntn[:, None]

  iid_e = iid_ref[...]
  icat_e = icat_ref[...]
  feat = jnp.concatenate(
      [age_ref[...], gen_ref[...], iid_e, icat_e, sm_id,
       sm_cat, iid_e * sm_id, icat_e * sm_cat, gnn], axis=1)
  h0 = jnp.maximum(
      jnp.dot(feat, wp0_ref[...], preferred_element_type=f32)
      + bp0_ref[...][None, :], 0.0)
  h1 = jnp.maximum(
      jnp.dot(h0, wp1_ref[...], preferred_element_type=f32)
      + bp1_ref[...][None, :], 0.0)
  logit = jnp.dot(h1, wp2_ref[...], preferred_element_type=f32) + bp2_ref[0]
  out_ref[...] = 1.0 / (1.0 + jnp.exp(-logit))


def _tc_call(idseq, nid_t, age_e, gen_e, iid_e, icat_e, sum_id, sum_cat,
             neigh_t, W_agg1, b_agg1, W_p0, b_p0, W_p1, b_p1, W_p2, b_p2):
  nblk = B // BS
  bcast = lambda shape: pl.BlockSpec(shape, lambda i: tuple(0 for _ in shape))
  return pl.pallas_call(
      _tc_body,
      grid=(nblk,),
      in_specs=[
          pl.BlockSpec((BS, L), lambda i: (i, 0)),
          pl.BlockSpec((NN, BS), lambda i: (0, i)),
          pl.BlockSpec((BS, KV), lambda i: (i, 0)),
          pl.BlockSpec((BS, KV), lambda i: (i, 0)),
          pl.BlockSpec((BS, KV), lambda i: (i, 0)),
          pl.BlockSpec((BS, KV), lambda i: (i, 0)),
          pl.BlockSpec((BS, KV), lambda i: (i, 0)),
          pl.BlockSpec((BS, KV), lambda i: (i, 0)),
          pl.BlockSpec((NN, BS, MEM), lambda i: (0, i, 0)),
          bcast((MEM, MEM)),
          bcast((MEM,)),
          bcast((5 * MEM, 256)),
          bcast((256,)),
          bcast((256, 128)),
          bcast((128,)),
          bcast((128, 1)),
          bcast((1,)),
      ],
      out_specs=pl.BlockSpec((BS, 1), lambda i: (i, 0)),
      out_shape=jax.ShapeDtypeStruct((B, 1), jnp.float32),
  )(idseq, nid_t, age_e, gen_e, iid_e, icat_e, sum_id, sum_cat, neigh_t,
    W_agg1, b_agg1, W_p0, b_p0, W_p1, b_p1, W_p2, b_p2)


def kernel(userid, user_age, user_gender, item_id, item_cate, item_id_seq,
           item_cate_seq, neighbor_ids, user_age_table, user_gender_table,
           item_id_table, item_cate_table, user_mem_0, user_mem_1,
           W_agg0, b_agg0, W_agg1, b_agg1,
           W_p0, b_p0, W_p1, b_p1, W_p2, b_p2):
  i32 = jnp.int32
  idseq = item_id_seq.astype(i32)
  cateff = jnp.where(idseq == 0, 0, item_cate_seq.astype(i32))
  nid = neighbor_ids.astype(i32)
  nid_t = nid.T

  (age_e, gen_e, iid_e, icat_e, sum_id, sum_cat, neigh_rows) = _make_sc_gather()(
      user_age_table, user_gender_table, item_id_table, item_cate_table,
      user_mem_1,
      user_age.astype(i32), user_gender.astype(i32),
      item_id.astype(i32), item_cate.astype(i32),
      idseq.reshape(-1), cateff.reshape(-1), nid_t.reshape(-1))

  out = _tc_call(
      idseq, nid_t, age_e, gen_e, iid_e, icat_e, sum_id, sum_cat,
      neigh_rows.reshape(NN, B, MEM),
      W_agg1, b_agg1, W_p0, b_p0, W_p1, b_p1, W_p2, b_p2)
  return out[:, 0]


# speedup vs baseline: 1.1290x; 1.0217x over previous
import functools

import jax
import jax.numpy as jnp
from jax import lax
from jax.experimental import pallas as pl
from jax.experimental.pallas import tpu as pltpu
from jax.experimental.pallas import tpu_sc as plsc

B = 4096
L = 50
NN = 20
KV = 32
MEM = 64

NC = 2
NS = 16
NW = NC * NS
PER_W = B // NW
SEQ_CB = 32
SEQ_NCHUNK = PER_W // SEQ_CB
SGL_CB = 128
NROWS_W = (B * NN) // NW
NEIGH_CB = 640
NEIGH_NCHUNK = NROWS_W // NEIGH_CB


@functools.lru_cache(maxsize=1)
def _make_sc_gather():
  mesh = plsc.VectorSubcoreMesh(
      core_axis_name="c", subcore_axis_name="s", num_cores=NC, num_subcores=NS)
  return functools.partial(
      pl.kernel,
      out_type=(
          jax.ShapeDtypeStruct((B, KV), jnp.float32),
          jax.ShapeDtypeStruct((B, KV), jnp.float32),
          jax.ShapeDtypeStruct((B, KV), jnp.float32),
          jax.ShapeDtypeStruct((B, KV), jnp.float32),
          jax.ShapeDtypeStruct((B, KV), jnp.float32),
          jax.ShapeDtypeStruct((B, KV), jnp.float32),
          jax.ShapeDtypeStruct((B * NN, MEM), jnp.float32),
      ),
      mesh=mesh,
      scratch_types=[
          pltpu.VMEM((SGL_CB,), jnp.int32),
          pltpu.VMEM((SGL_CB, KV), jnp.float32),
          pltpu.VMEM((SEQ_CB * L,), jnp.int32),
          pltpu.VMEM((SEQ_CB * L, KV), jnp.float32),
          pltpu.VMEM((SEQ_CB, KV), jnp.float32),
          pltpu.VMEM((NEIGH_CB,), jnp.int32),
          pltpu.VMEM((NEIGH_CB, MEM), jnp.float32),
          pltpu.SemaphoreType.DMA,
      ],
      compiler_params=pltpu.CompilerParams(use_tc_tiling_on_sc=False),
  )(_sc_gather_body)


def _sc_gather_body(age_tab, gen_tab, iid_tab, icat_tab, mem1_tab,
                    age_idx, gen_idx, iid_idx, icat_idx,
                    idseq, cateff, nidx,
                    age_out, gen_out, iid_out, icat_out, sumid_out, sumcat_out,
                    neigh_out,
                    sidx_v, srows_v, qidx_v, qrows_v, qsum_v, nidx_v, nrows_v,
                    sem):
  wid = lax.axis_index("s") * NC + lax.axis_index("c")
  base = wid * PER_W

  for tab, idx_hbm, out_hbm in (
      (age_tab, age_idx, age_out),
      (gen_tab, gen_idx, gen_out),
      (iid_tab, iid_idx, iid_out),
      (icat_tab, icat_idx, icat_out),
  ):
    for c in range(PER_W // SGL_CB):
      off = base + c * SGL_CB
      pltpu.sync_copy(idx_hbm.at[pl.ds(off, SGL_CB)], sidx_v)
      pltpu.async_copy(tab.at[sidx_v], srows_v, sem).wait()
      pltpu.sync_copy(srows_v, out_hbm.at[pl.ds(off, SGL_CB)])

  for tab, idx_hbm, out_hbm in ((iid_tab, idseq, sumid_out),
                                (icat_tab, cateff, sumcat_out)):
    def seq_chunk(c, tab=tab, idx_hbm=idx_hbm, out_hbm=out_hbm):
      roff = base * L + c * (SEQ_CB * L)
      pltpu.sync_copy(idx_hbm.at[pl.ds(roff, SEQ_CB * L)], qidx_v)
      pltpu.async_copy(tab.at[qidx_v], qrows_v, sem).wait()

      def accum_b(b, _):
        r0 = b * L

        def accum_l(l, accs):
          a0, a1 = accs
          return (a0 + qrows_v[r0 + l, pl.ds(0, 16)],
                  a1 + qrows_v[r0 + l, pl.ds(16, 16)])

        z = jnp.zeros((16,), jnp.float32)
        a0, a1 = lax.fori_loop(0, L, accum_l, (z, z), unroll=10)
        qsum_v[b, pl.ds(0, 16)] = a0
        qsum_v[b, pl.ds(16, 16)] = a1
        return 0

      lax.fori_loop(0, SEQ_CB, accum_b, 0)
      pltpu.sync_copy(qsum_v, out_hbm.at[pl.ds(base + c * SEQ_CB, SEQ_CB)])
      return 0

    lax.fori_loop(0, SEQ_NCHUNK, lambda c, _, f=seq_chunk: f(c), 0)

  nbase = wid * NROWS_W

  def neigh_chunk(c, _):
    roff = nbase + c * NEIGH_CB
    pltpu.sync_copy(nidx.at[pl.ds(roff, NEIGH_CB)], nidx_v)
    pltpu.async_copy(mem1_tab.at[nidx_v], nrows_v, sem).wait()
    pltpu.sync_copy(nrows_v, neigh_out.at[pl.ds(roff, NEIGH_CB)])
    return 0

  lax.fori_loop(0, NEIGH_NCHUNK, neigh_chunk, 0)


BS = 256


def _tc_body(idseq_ref, nidt_ref, age_ref, gen_ref, iid_ref, icat_ref,
             sumid_ref, sumcat_ref, neigh_ref,
             wagg_ref, bagg_ref, wp0_ref, bp0_ref, wp1_ref, bp1_ref,
             wp2_ref, bp2_ref, out_ref):
  f32 = jnp.float32
  cnt = jnp.sum((idseq_ref[...] != 0).astype(f32), axis=1)
  den = jnp.maximum(cnt, 1.0)[:, None]
  sm_id = sumid_ref[...] / den
  sm_cat = sumcat_ref[...] / den

  nm = (nidt_ref[...] != 0).astype(f32)
  cntn = jnp.maximum(jnp.sum(nm, axis=0), 1.0)
  W = wagg_ref[...]
  bb = bagg_ref[...]
  acc = jnp.zeros((BS, MEM), f32)
  for n in range(NN):
    h = jnp.tanh(
        jnp.dot(neigh_ref[n], W, preferred_element_type=f32)
        + bb[None, :])
    acc = acc + h * nm[n][:, None]
  gnn = acc / cntn[:, None]

  iid_e = iid_ref[...]
  icat_e = icat_ref[...]
  feat = jnp.concatenate(
      [age_ref[...], gen_ref[...], iid_e, icat_e, sm_id,
       sm_cat, iid_e * sm_id, icat_e * sm_cat, gnn], axis=1)
  h0 = jnp.maximum(
      jnp.dot(feat, wp0_ref[...], preferred_element_type=f32)
      + bp0_ref[...][None, :], 0.0)
  h1 = jnp.maximum(
      jnp.dot(h0, wp1_ref[...], preferred_element_type=f32)
      + bp1_ref[...][None, :], 0.0)
  logit = jnp.dot(h1, wp2_ref[...], preferred_element_type=f32) + bp2_ref[0]
  out_ref[...] = 1.0 / (1.0 + jnp.exp(-logit))


def _tc_call(idseq, nid_t, age_e, gen_e, iid_e, icat_e, sum_id, sum_cat,
             neigh_t, W_agg1, b_agg1, W_p0, b_p0, W_p1, b_p1, W_p2, b_p2):
  nblk = B // BS
  bcast = lambda shape: pl.BlockSpec(shape, lambda i: tuple(0 for _ in shape))
  return pl.pallas_call(
      _tc_body,
      grid=(nblk,),
      in_specs=[
          pl.BlockSpec((BS, L), lambda i: (i, 0)),
          pl.BlockSpec((NN, BS), lambda i: (0, i)),
          pl.BlockSpec((BS, KV), lambda i: (i, 0)),
          pl.BlockSpec((BS, KV), lambda i: (i, 0)),
          pl.BlockSpec((BS, KV), lambda i: (i, 0)),
          pl.BlockSpec((BS, KV), lambda i: (i, 0)),
          pl.BlockSpec((BS, KV), lambda i: (i, 0)),
          pl.BlockSpec((BS, KV), lambda i: (i, 0)),
          pl.BlockSpec((NN, BS, MEM), lambda i: (0, i, 0)),
          bcast((MEM, MEM)),
          bcast((MEM,)),
          bcast((5 * MEM, 256)),
          bcast((256,)),
          bcast((256, 128)),
          bcast((128,)),
          bcast((128, 1)),
          bcast((1,)),
      ],
      out_specs=pl.BlockSpec((BS, 1), lambda i: (i, 0)),
      out_shape=jax.ShapeDtypeStruct((B, 1), jnp.float32),
  )(idseq, nid_t, age_e, gen_e, iid_e, icat_e, sum_id, sum_cat, neigh_t,
    W_agg1, b_agg1, W_p0, b_p0, W_p1, b_p1, W_p2, b_p2)


def kernel(userid, user_age, user_gender, item_id, item_cate, item_id_seq,
           item_cate_seq, neighbor_ids, user_age_table, user_gender_table,
           item_id_table, item_cate_table, user_mem_0, user_mem_1,
           W_agg0, b_agg0, W_agg1, b_agg1,
           W_p0, b_p0, W_p1, b_p1, W_p2, b_p2):
  i32 = jnp.int32
  idseq = item_id_seq.astype(i32)
  cateff = jnp.where(idseq == 0, 0, item_cate_seq.astype(i32))
  nid = neighbor_ids.astype(i32)
  nid_t = nid.T

  (age_e, gen_e, iid_e, icat_e, sum_id, sum_cat, neigh_rows) = _make_sc_gather()(
      user_age_table, user_gender_table, item_id_table, item_cate_table,
      user_mem_1,
      user_age.astype(i32), user_gender.astype(i32),
      item_id.astype(i32), item_cate.astype(i32),
      idseq.reshape(-1), cateff.reshape(-1), nid_t.reshape(-1))

  out = _tc_call(
      idseq, nid_t, age_e, gen_e, iid_e, icat_e, sum_id, sum_cat,
      neigh_rows.reshape(NN, B, MEM),
      W_agg1, b_agg1, W_p0, b_p0, W_p1, b_p1, W_p2, b_p2)
  return out[:, 0]
